# baseline (speedup 1.0000x reference)
import jax
import jax.numpy as jnp
from jax.experimental import pallas as pl
from jax.experimental.pallas import tpu as pltpu

_NB = 4
_NO = 32
_TM = 1024
_TK = 2048


def _h_kernel(af_ref, wt_ref, b_ref, out_ref):
    out_ref[0] = (
        jnp.dot(af_ref[...], wt_ref[0], preferred_element_type=jnp.float32)
        + b_ref[0]
    ).astype(out_ref.dtype)


def _mm_kernel(bi_ref, h_ref, out_ref):
    k = pl.program_id(1)

    @pl.when(k == 0)
    def _():
        out_ref[...] = jnp.zeros_like(out_ref)

    out_ref[...] += jnp.dot(
        bi_ref[...].astype(jnp.bfloat16),
        h_ref[...],
        preferred_element_type=jnp.float32,
    )


def kernel(atom_features, bond_info, W, b):
    n, f = atom_features.shape
    wt = W.reshape(_NB, _NO, f).transpose(0, 2, 1)
    b2 = b.reshape(_NB, 1, _NO)

    h3 = pl.pallas_call(
        _h_kernel,
        grid=(_NB,),
        in_specs=[
            pl.BlockSpec((n, f), lambda t: (0, 0)),
            pl.BlockSpec((1, f, _NO), lambda t: (t, 0, 0)),
            pl.BlockSpec((1, 1, _NO), lambda t: (t, 0, 0)),
        ],
        out_specs=pl.BlockSpec((1, n, _NO), lambda t: (t, 0, 0)),
        out_shape=jax.ShapeDtypeStruct((_NB, n, _NO), jnp.bfloat16),
    )(atom_features, wt, b2)
    h2 = h3.reshape(_NB * n, _NO)

    grid = (n // _TM, (_NB * n) // _TK)
    out = pl.pallas_call(
        _mm_kernel,
        grid=grid,
        in_specs=[
            pl.BlockSpec((_TM, _TK), lambda i, k: (i, k)),
            pl.BlockSpec((_TK, _NO), lambda i, k: (k, 0)),
        ],
        out_specs=pl.BlockSpec((_TM, _NO), lambda i, k: (i, 0)),
        out_shape=jax.ShapeDtypeStruct((n, _NO), jnp.float32),
        compiler_params=pltpu.CompilerParams(
            dimension_semantics=("parallel", "arbitrary"),
        ),
    )(bond_info, h2)
    return out
